# baseline (speedup 1.0000x reference)
import functools

import jax
import jax.numpy as jnp
from jax import lax
from jax.experimental import pallas as pl
from jax.experimental.pallas import tpu as pltpu, tpu_sc as plsc

NC, NS = 2, 16
NW = NC * NS
CH = 128
CW = 16


def _tc_proj(x_ref, wlT_ref, wrT_ref, bl_ref, z_ref, r_ref):
    xv = x_ref[...]
    z_ref[...] = jnp.dot(xv, wlT_ref[...], preferred_element_type=jnp.float32)
    r_ref[...] = (jnp.dot(xv, wrT_ref[...], preferred_element_type=jnp.float32)
                  + bl_ref[...])


def _tc_mid(part_ref, cntp_ref, r_ref, wlT_ref, wrT_ref, bl_ref, z_ref, rn_ref):
    n = r_ref.shape[0]
    cnt = cntp_ref[0, :n, 0:1] + cntp_ref[1, :n, 0:1]
    inv = 1.0 / jnp.maximum(cnt, 1.0)
    agg = part_ref[0, :n, :] + part_ref[1, :n, :]
    h = jnp.maximum(agg * inv + r_ref[...], 0.0)
    z_ref[...] = jnp.dot(h, wlT_ref[...], preferred_element_type=jnp.float32)
    rn_ref[...] = (jnp.dot(h, wrT_ref[...], preferred_element_type=jnp.float32)
                   + bl_ref[...])


def _tc_fin(part_ref, cntp_ref, r_ref, o_ref):
    n = r_ref.shape[0]
    cnt = cntp_ref[0, :n, 0:1] + cntp_ref[1, :n, 0:1]
    inv = 1.0 / jnp.maximum(cnt, 1.0)
    agg = part_ref[0, :n, :] + part_ref[1, :n, :]
    o_ref[...] = agg * inv + r_ref[...]


NBUF = 8
LOOK = 4


def _sc_agg(n, npad, k, dh, with_cnt, *refs):
    if with_cnt:
        (z_hbm, src_hbm, dst_hbm, ones_hbm, zeros_hbm, zerosc_hbm,
         part_hbm, cntp_hbm,
         src_v, dst_v, rows_v, ones_v, acc_sh, cnt_sh, *sems) = refs
    else:
        (z_hbm, src_hbm, dst_hbm, zeros_hbm,
         part_hbm,
         src_v, dst_v, rows_v, acc_sh, *sems) = refs
    gsems = sems[:NBUF]
    ssems = sems[NBUF:2 * NBUF]
    csem = sems[2 * NBUF]
    c = lax.axis_index("c")
    s = lax.axis_index("s")
    blk = c * NS + s
    zrows = npad // NS
    zr = s * zrows
    pltpu.sync_copy(zeros_hbm.at[pl.ds(zr, zrows)], acc_sh.at[pl.ds(zr, zrows)])
    if with_cnt:
        pltpu.sync_copy(zerosc_hbm.at[pl.ds(zr, zrows)],
                        cnt_sh.at[pl.ds(zr, zrows)])
        pltpu.sync_copy(ones_hbm, ones_v)
    pltpu.sync_copy(src_hbm.at[blk], src_v)
    pltpu.sync_copy(dst_hbm.at[blk], dst_v)
    plsc.subcore_barrier()

    for b in range(LOOK):
        pltpu.async_copy(z_hbm.at[src_v.at[b]], rows_v.at[b], gsems[b])

    def body(jo, carry):
        for b8 in range(NBUF):
            j = jo * NBUF + b8
            b = b8
            bn = (b8 + LOOK) % NBUF
            pltpu.make_async_copy(z_hbm.at[src_v.at[j]], rows_v.at[b],
                                  gsems[b]).wait()
            pltpu.async_copy(rows_v.at[b], acc_sh.at[dst_v.at[j]],
                             ssems[b], add=True)
            if with_cnt:
                pltpu.async_copy(ones_v, cnt_sh.at[dst_v.at[j]],
                                 csem, add=True)
            jn = j + LOOK

            @pl.when(jnp.logical_and(jn < k, j >= LOOK))
            def _():
                pltpu.make_async_copy(rows_v.at[bn], acc_sh.at[dst_v.at[j]],
                                      ssems[bn]).wait()

            @pl.when(jn < k)
            def _():
                pltpu.async_copy(z_hbm.at[src_v.at[jn]], rows_v.at[bn],
                                 gsems[bn])
        return carry

    lax.fori_loop(0, k // NBUF, body, 0)
    for b in range(NBUF):
        pltpu.make_async_copy(rows_v.at[b], acc_sh.at[dst_v.at[0]],
                              ssems[b]).wait()
    if with_cnt:
        def cdrain(j, carry):
            pltpu.make_async_copy(ones_v, cnt_sh.at[dst_v.at[0]],
                                  csem).wait()
            return carry
        lax.fori_loop(0, k, cdrain, 0)
    plsc.subcore_barrier()
    pltpu.sync_copy(acc_sh.at[pl.ds(zr, zrows)],
                    part_hbm.at[c, pl.ds(zr, zrows)])
    if with_cnt:
        pltpu.sync_copy(cnt_sh.at[pl.ds(zr, zrows)],
                        cntp_hbm.at[c, pl.ds(zr, zrows)])


def kernel(x, edge_index, Wl0, bl0, Wr0, Wl1, bl1, Wr1, Wl2, bl2, Wr2):
    n, d_in = x.shape
    dh = Wl0.shape[0]
    e = edge_index.shape[1]
    k = -(-e // (NW * CH))
    k = -(-k // NBUF) * NBUF
    epad = NW * k * CH
    npad = NS * (-(-(n + 1) // (NS * 8)) * 8)

    src = edge_index[0]
    dst = edge_index[1]
    pad = epad - e
    srcp = jnp.concatenate([src, jnp.zeros((pad,), jnp.int32)]).reshape(NW, k, CH)
    dstp = jnp.concatenate([dst, jnp.full((pad,), n, jnp.int32)]).reshape(NW, k, CH)
    ones = jnp.ones((CH, CW), jnp.float32)
    zeros = jnp.zeros((npad, dh), jnp.float32)
    zerosc = jnp.zeros((npad, CW), jnp.float32)
    f32 = jnp.float32

    proj = pl.pallas_call(
        _tc_proj,
        out_shape=(jax.ShapeDtypeStruct((n, dh), f32),
                   jax.ShapeDtypeStruct((n, dh), f32)),
    )
    mid = pl.pallas_call(
        _tc_mid,
        out_shape=(jax.ShapeDtypeStruct((n, dh), f32),
                   jax.ShapeDtypeStruct((n, dh), f32)),
    )
    fin = pl.pallas_call(
        _tc_fin,
        out_shape=jax.ShapeDtypeStruct((n, dh), f32),
    )

    mesh = plsc.VectorSubcoreMesh(core_axis_name="c", subcore_axis_name="s",
                                  num_cores=NC, num_subcores=NS)
    sc_params = pltpu.CompilerParams(use_tc_tiling_on_sc=False)
    common_scratch = [
        pltpu.VMEM((k, CH), jnp.int32),
        pltpu.VMEM((k, CH), jnp.int32),
        pltpu.VMEM((NBUF, CH, dh), f32),
    ]
    sem_scratch = [pltpu.SemaphoreType.DMA] * (2 * NBUF + 1)
    agg_cnt = pl.kernel(
        functools.partial(_sc_agg, n, npad, k, dh, True),
        out_type=(jax.ShapeDtypeStruct((NC, npad, dh), f32),
                  jax.ShapeDtypeStruct((NC, npad, CW), f32)),
        mesh=mesh,
        scratch_types=common_scratch + [
            pltpu.VMEM((CH, CW), f32),
            pltpu.VMEM_SHARED((npad, dh), f32),
            pltpu.VMEM_SHARED((npad, CW), f32),
        ] + sem_scratch,
        compiler_params=sc_params,
    )
    agg = pl.kernel(
        functools.partial(_sc_agg, n, npad, k, dh, False),
        out_type=jax.ShapeDtypeStruct((NC, npad, dh), f32),
        mesh=mesh,
        scratch_types=common_scratch + [
            pltpu.VMEM_SHARED((npad, dh), f32),
        ] + sem_scratch,
        compiler_params=sc_params,
    )

    bl0r = bl0.reshape(1, dh)
    bl1r = bl1.reshape(1, dh)
    bl2r = bl2.reshape(1, dh)

    z0, r0 = proj(x, Wl0.T, Wr0.T, bl0r)
    part0, cntp = agg_cnt(z0, srcp, dstp, ones, zeros, zerosc)
    z1, r1 = mid(part0, cntp, r0, Wl1.T, Wr1.T, bl1r)
    part1 = agg(z1, srcp, dstp, zeros)
    z2, r2 = mid(part1, cntp, r1, Wl2.T, Wr2.T, bl2r)
    part2 = agg(z2, srcp, dstp, zeros)
    return fin(part2, cntp, r2)


# speedup vs baseline: 1.9909x; 1.9909x over previous
import functools

import jax
import jax.numpy as jnp
from jax import lax
from jax.experimental import pallas as pl
from jax.experimental.pallas import tpu as pltpu, tpu_sc as plsc

NC, NS = 2, 16
NW = NC * NS
CH = 128
CW = 16


def _tc_proj(x_ref, wlT_ref, wrT_ref, bl_ref, z_ref, r_ref):
    n = r_ref.shape[0]
    npad, dh = z_ref.shape
    xv = x_ref[...]
    z_ref[0:n, :] = jnp.dot(xv, wlT_ref[...], preferred_element_type=jnp.float32)
    z_ref[n:npad, :] = jnp.zeros((npad - n, dh), jnp.float32)
    r_ref[...] = (jnp.dot(xv, wrT_ref[...], preferred_element_type=jnp.float32)
                  + bl_ref[...])


def _tc_mid(part_ref, cntp_ref, r_ref, wlT_ref, wrT_ref, bl_ref, z_ref, rn_ref):
    n = r_ref.shape[0]
    npad, dh = z_ref.shape
    cnt = cntp_ref[0, :n, 0:1] + cntp_ref[1, :n, 0:1]
    inv = 1.0 / jnp.maximum(cnt, 1.0)
    agg = part_ref[0, :n, :] + part_ref[1, :n, :]
    h = jnp.maximum(agg * inv + r_ref[...], 0.0)
    z_ref[0:n, :] = jnp.dot(h, wlT_ref[...], preferred_element_type=jnp.float32)
    z_ref[n:npad, :] = jnp.zeros((npad - n, dh), jnp.float32)
    rn_ref[...] = (jnp.dot(h, wrT_ref[...], preferred_element_type=jnp.float32)
                   + bl_ref[...])


def _tc_fin(part_ref, cntp_ref, r_ref, o_ref):
    n = r_ref.shape[0]
    cnt = cntp_ref[0, :n, 0:1] + cntp_ref[1, :n, 0:1]
    inv = 1.0 / jnp.maximum(cnt, 1.0)
    agg = part_ref[0, :n, :] + part_ref[1, :n, :]
    o_ref[...] = agg * inv + r_ref[...]


NBUF = 8
LOOK = 4


def _sc_agg(n, npad, k, dh, with_cnt, *refs):
    if with_cnt:
        (z_hbm, src_hbm, dst_hbm, ones_hbm, zeros_hbm, zerosc_hbm,
         part_hbm, cntp_hbm,
         src_v, dst_v, rows_v, ones_v, acc_sh, z_sh, cnt_sh, *sems) = refs
    else:
        (z_hbm, src_hbm, dst_hbm, zeros_hbm,
         part_hbm,
         src_v, dst_v, rows_v, acc_sh, z_sh, *sems) = refs
    gsems = sems[:NBUF]
    ssems = sems[NBUF:2 * NBUF]
    csem = sems[2 * NBUF]
    c = lax.axis_index("c")
    s = lax.axis_index("s")
    blk = c * NS + s
    zrows = npad // NS
    zr = s * zrows
    pltpu.sync_copy(zeros_hbm.at[pl.ds(zr, zrows)], acc_sh.at[pl.ds(zr, zrows)])
    pltpu.sync_copy(z_hbm.at[pl.ds(zr, zrows)], z_sh.at[pl.ds(zr, zrows)])
    if with_cnt:
        pltpu.sync_copy(zerosc_hbm.at[pl.ds(zr, zrows)],
                        cnt_sh.at[pl.ds(zr, zrows)])
        pltpu.sync_copy(ones_hbm, ones_v)
    pltpu.sync_copy(src_hbm.at[blk], src_v)
    pltpu.sync_copy(dst_hbm.at[blk], dst_v)
    plsc.subcore_barrier()

    for b in range(LOOK):
        pltpu.async_copy(z_sh.at[src_v.at[b]], rows_v.at[b], gsems[b])

    def body(jo, carry):
        for b8 in range(NBUF):
            j = jo * NBUF + b8
            b = b8
            bn = (b8 + LOOK) % NBUF
            pltpu.make_async_copy(z_sh.at[src_v.at[j]], rows_v.at[b],
                                  gsems[b]).wait()
            pltpu.async_copy(rows_v.at[b], acc_sh.at[dst_v.at[j]],
                             ssems[b], add=True)
            if with_cnt:
                pltpu.async_copy(ones_v, cnt_sh.at[dst_v.at[j]],
                                 csem, add=True)
            jn = j + LOOK

            @pl.when(jnp.logical_and(jn < k, j >= LOOK))
            def _():
                pltpu.make_async_copy(rows_v.at[bn], acc_sh.at[dst_v.at[j]],
                                      ssems[bn]).wait()

            @pl.when(jn < k)
            def _():
                pltpu.async_copy(z_sh.at[src_v.at[jn]], rows_v.at[bn],
                                 gsems[bn])
        return carry

    lax.fori_loop(0, k // NBUF, body, 0)
    for b in range(NBUF):
        pltpu.make_async_copy(rows_v.at[b], acc_sh.at[dst_v.at[0]],
                              ssems[b]).wait()
    if with_cnt:
        def cdrain(j, carry):
            pltpu.make_async_copy(ones_v, cnt_sh.at[dst_v.at[0]],
                                  csem).wait()
            return carry
        lax.fori_loop(0, k, cdrain, 0)
    plsc.subcore_barrier()
    pltpu.sync_copy(acc_sh.at[pl.ds(zr, zrows)],
                    part_hbm.at[c, pl.ds(zr, zrows)])
    if with_cnt:
        pltpu.sync_copy(cnt_sh.at[pl.ds(zr, zrows)],
                        cntp_hbm.at[c, pl.ds(zr, zrows)])


def kernel(x, edge_index, Wl0, bl0, Wr0, Wl1, bl1, Wr1, Wl2, bl2, Wr2):
    n, d_in = x.shape
    dh = Wl0.shape[0]
    e = edge_index.shape[1]
    k = -(-e // (NW * CH))
    k = -(-k // NBUF) * NBUF
    epad = NW * k * CH
    npad = NS * (-(-(n + 1) // (NS * 8)) * 8)

    src = edge_index[0]
    dst = edge_index[1]
    pad = epad - e
    srcp = jnp.concatenate([src, jnp.zeros((pad,), jnp.int32)]).reshape(NW, k, CH)
    dstp = jnp.concatenate([dst, jnp.full((pad,), n, jnp.int32)]).reshape(NW, k, CH)
    ones = jnp.ones((CH, CW), jnp.float32)
    zeros = jnp.zeros((npad, dh), jnp.float32)
    zerosc = jnp.zeros((npad, CW), jnp.float32)
    f32 = jnp.float32

    proj = pl.pallas_call(
        _tc_proj,
        out_shape=(jax.ShapeDtypeStruct((npad, dh), f32),
                   jax.ShapeDtypeStruct((n, dh), f32)),
    )
    mid = pl.pallas_call(
        _tc_mid,
        out_shape=(jax.ShapeDtypeStruct((npad, dh), f32),
                   jax.ShapeDtypeStruct((n, dh), f32)),
    )
    fin = pl.pallas_call(
        _tc_fin,
        out_shape=jax.ShapeDtypeStruct((n, dh), f32),
    )

    mesh = plsc.VectorSubcoreMesh(core_axis_name="c", subcore_axis_name="s",
                                  num_cores=NC, num_subcores=NS)
    sc_params = pltpu.CompilerParams(use_tc_tiling_on_sc=False)
    common_scratch = [
        pltpu.VMEM((k, CH), jnp.int32),
        pltpu.VMEM((k, CH), jnp.int32),
        pltpu.VMEM((NBUF, CH, dh), f32),
    ]
    sem_scratch = [pltpu.SemaphoreType.DMA] * (2 * NBUF + 1)
    agg_cnt = pl.kernel(
        functools.partial(_sc_agg, n, npad, k, dh, True),
        out_type=(jax.ShapeDtypeStruct((NC, npad, dh), f32),
                  jax.ShapeDtypeStruct((NC, npad, CW), f32)),
        mesh=mesh,
        scratch_types=common_scratch + [
            pltpu.VMEM((CH, CW), f32),
            pltpu.VMEM_SHARED((npad, dh), f32),
            pltpu.VMEM_SHARED((npad, dh), f32),
            pltpu.VMEM_SHARED((npad, CW), f32),
        ] + sem_scratch,
        compiler_params=sc_params,
    )
    agg = pl.kernel(
        functools.partial(_sc_agg, n, npad, k, dh, False),
        out_type=jax.ShapeDtypeStruct((NC, npad, dh), f32),
        mesh=mesh,
        scratch_types=common_scratch + [
            pltpu.VMEM_SHARED((npad, dh), f32),
            pltpu.VMEM_SHARED((npad, dh), f32),
        ] + sem_scratch,
        compiler_params=sc_params,
    )

    bl0r = bl0.reshape(1, dh)
    bl1r = bl1.reshape(1, dh)
    bl2r = bl2.reshape(1, dh)

    z0, r0 = proj(x, Wl0.T, Wr0.T, bl0r)
    part0, cntp = agg_cnt(z0, srcp, dstp, ones, zeros, zerosc)
    z1, r1 = mid(part0, cntp, r0, Wl1.T, Wr1.T, bl1r)
    part1 = agg(z1, srcp, dstp, zeros)
    z2, r2 = mid(part1, cntp, r1, Wl2.T, Wr2.T, bl2r)
    part2 = agg(z2, srcp, dstp, zeros)
    return fin(part2, cntp, r2)


# speedup vs baseline: 2.0731x; 1.0413x over previous
import functools

import jax
import jax.numpy as jnp
from jax import lax
from jax.experimental import pallas as pl
from jax.experimental.pallas import tpu as pltpu, tpu_sc as plsc

NC, NS = 2, 16
NW = NC * NS
CH = 128
CW = 16


def _tc_proj(x_ref, wlT_ref, wrT_ref, bl_ref, z_ref, r_ref):
    n = r_ref.shape[0]
    npad, dh = z_ref.shape
    xv = x_ref[...]
    z_ref[0:n, :] = jnp.dot(xv, wlT_ref[...], preferred_element_type=jnp.float32)
    z_ref[n:npad, :] = jnp.zeros((npad - n, dh), jnp.float32)
    r_ref[...] = (jnp.dot(xv, wrT_ref[...], preferred_element_type=jnp.float32)
                  + bl_ref[...])


def _tc_mid(part_ref, cntp_ref, r_ref, wlT_ref, wrT_ref, bl_ref, z_ref, rn_ref):
    n = r_ref.shape[0]
    npad, dh = z_ref.shape
    cnt = cntp_ref[0, :n, 0:1] + cntp_ref[1, :n, 0:1]
    inv = 1.0 / jnp.maximum(cnt, 1.0)
    agg = part_ref[0, :n, :] + part_ref[1, :n, :]
    h = jnp.maximum(agg * inv + r_ref[...], 0.0)
    z_ref[0:n, :] = jnp.dot(h, wlT_ref[...], preferred_element_type=jnp.float32)
    z_ref[n:npad, :] = jnp.zeros((npad - n, dh), jnp.float32)
    rn_ref[...] = (jnp.dot(h, wrT_ref[...], preferred_element_type=jnp.float32)
                   + bl_ref[...])


def _tc_fin(part_ref, cntp_ref, r_ref, o_ref):
    n = r_ref.shape[0]
    cnt = cntp_ref[0, :n, 0:1] + cntp_ref[1, :n, 0:1]
    inv = 1.0 / jnp.maximum(cnt, 1.0)
    agg = part_ref[0, :n, :] + part_ref[1, :n, :]
    o_ref[...] = agg * inv + r_ref[...]


NBUF = 8
LOOK = 4


def _sc_agg(n, npad, k, dh, with_cnt, *refs):
    if with_cnt:
        (z_hbm, ei_hbm, ones_hbm, zeros_hbm, zerosc_hbm,
         part_hbm, cntp_hbm,
         src_v, dst_v, rows_v, ones_v, acc_sh, z_sh, cnt_sh, *sems) = refs
    else:
        (z_hbm, ei_hbm, zeros_hbm,
         part_hbm,
         src_v, dst_v, rows_v, acc_sh, z_sh, *sems) = refs
    gsems = sems[:NBUF]
    ssems = sems[NBUF:2 * NBUF]
    csem = sems[2 * NBUF]
    c = lax.axis_index("c")
    s = lax.axis_index("s")
    blk = c * NS + s
    zrows = npad // NS
    zr = s * zrows
    ct = ei_hbm.shape[1]
    kbase = ct // NW
    extras = ct - NW * kbase
    pltpu.sync_copy(zeros_hbm.at[pl.ds(zr, zrows)], acc_sh.at[pl.ds(zr, zrows)])
    pltpu.sync_copy(z_hbm.at[pl.ds(zr, zrows)], z_sh.at[pl.ds(zr, zrows)])
    if with_cnt:
        pltpu.sync_copy(zerosc_hbm.at[pl.ds(zr, zrows)],
                        cnt_sh.at[pl.ds(zr, zrows)])
        pltpu.sync_copy(ones_hbm, ones_v)
    pltpu.sync_copy(ei_hbm.at[0, pl.ds(blk * kbase, kbase)],
                    src_v.at[pl.ds(0, kbase)])
    pltpu.sync_copy(ei_hbm.at[1, pl.ds(blk * kbase, kbase)],
                    dst_v.at[pl.ds(0, kbase)])
    zero16 = jnp.zeros((16,), jnp.int32)
    pad16 = jnp.full((16,), n, jnp.int32)
    for row in range(kbase, k):
        for i in range(CH // 16):
            src_v[row, pl.ds(i * 16, 16)] = zero16
            dst_v[row, pl.ds(i * 16, 16)] = pad16

    @pl.when(blk < extras)
    def _():
        pltpu.sync_copy(ei_hbm.at[0, pl.ds(NW * kbase + blk, 1)],
                        src_v.at[pl.ds(kbase, 1)])
        pltpu.sync_copy(ei_hbm.at[1, pl.ds(NW * kbase + blk, 1)],
                        dst_v.at[pl.ds(kbase, 1)])

    plsc.subcore_barrier()

    for b in range(LOOK):
        pltpu.async_copy(z_sh.at[src_v.at[b]], rows_v.at[b], gsems[b])

    def body(jo, carry):
        for b8 in range(NBUF):
            j = jo * NBUF + b8
            b = b8
            bn = (b8 + LOOK) % NBUF
            pltpu.make_async_copy(z_sh.at[src_v.at[j]], rows_v.at[b],
                                  gsems[b]).wait()
            pltpu.async_copy(rows_v.at[b], acc_sh.at[dst_v.at[j]],
                             ssems[b], add=True)
            if with_cnt:
                pltpu.async_copy(ones_v, cnt_sh.at[dst_v.at[j]],
                                 csem, add=True)
            jn = j + LOOK

            @pl.when(jnp.logical_and(jn < k, j >= LOOK))
            def _():
                pltpu.make_async_copy(rows_v.at[bn], acc_sh.at[dst_v.at[j]],
                                      ssems[bn]).wait()

            @pl.when(jn < k)
            def _():
                pltpu.async_copy(z_sh.at[src_v.at[jn]], rows_v.at[bn],
                                 gsems[bn])
        return carry

    lax.fori_loop(0, k // NBUF, body, 0)
    for b in range(NBUF):
        pltpu.make_async_copy(rows_v.at[b], acc_sh.at[dst_v.at[0]],
                              ssems[b]).wait()
    if with_cnt:
        def cdrain(j, carry):
            pltpu.make_async_copy(ones_v, cnt_sh.at[dst_v.at[0]],
                                  csem).wait()
            return carry
        lax.fori_loop(0, k, cdrain, 0)
    plsc.subcore_barrier()
    pltpu.sync_copy(acc_sh.at[pl.ds(zr, zrows)],
                    part_hbm.at[c, pl.ds(zr, zrows)])
    if with_cnt:
        pltpu.sync_copy(cnt_sh.at[pl.ds(zr, zrows)],
                        cntp_hbm.at[c, pl.ds(zr, zrows)])


def kernel(x, edge_index, Wl0, bl0, Wr0, Wl1, bl1, Wr1, Wl2, bl2, Wr2):
    n, d_in = x.shape
    dh = Wl0.shape[0]
    e = edge_index.shape[1]
    ct = e // CH
    k = ct // NW + (1 if ct % NW else 0)
    k = -(-k // NBUF) * NBUF
    npad = NS * (-(-(n + 1) // (NS * 8)) * 8)

    ei = edge_index.reshape(2, ct, CH)
    ones = jnp.ones((CH, CW), jnp.float32)
    zeros = jnp.zeros((npad, dh), jnp.float32)
    zerosc = jnp.zeros((npad, CW), jnp.float32)
    f32 = jnp.float32

    proj = pl.pallas_call(
        _tc_proj,
        out_shape=(jax.ShapeDtypeStruct((npad, dh), f32),
                   jax.ShapeDtypeStruct((n, dh), f32)),
    )
    mid = pl.pallas_call(
        _tc_mid,
        out_shape=(jax.ShapeDtypeStruct((npad, dh), f32),
                   jax.ShapeDtypeStruct((n, dh), f32)),
    )
    fin = pl.pallas_call(
        _tc_fin,
        out_shape=jax.ShapeDtypeStruct((n, dh), f32),
    )

    mesh = plsc.VectorSubcoreMesh(core_axis_name="c", subcore_axis_name="s",
                                  num_cores=NC, num_subcores=NS)
    sc_params = pltpu.CompilerParams(use_tc_tiling_on_sc=False)
    common_scratch = [
        pltpu.VMEM((k, CH), jnp.int32),
        pltpu.VMEM((k, CH), jnp.int32),
        pltpu.VMEM((NBUF, CH, dh), f32),
    ]
    sem_scratch = [pltpu.SemaphoreType.DMA] * (2 * NBUF + 1)
    agg_cnt = pl.kernel(
        functools.partial(_sc_agg, n, npad, k, dh, True),
        out_type=(jax.ShapeDtypeStruct((NC, npad, dh), f32),
                  jax.ShapeDtypeStruct((NC, npad, CW), f32)),
        mesh=mesh,
        scratch_types=common_scratch + [
            pltpu.VMEM((CH, CW), f32),
            pltpu.VMEM_SHARED((npad, dh), f32),
            pltpu.VMEM_SHARED((npad, dh), f32),
            pltpu.VMEM_SHARED((npad, CW), f32),
        ] + sem_scratch,
        compiler_params=sc_params,
    )
    agg = pl.kernel(
        functools.partial(_sc_agg, n, npad, k, dh, False),
        out_type=jax.ShapeDtypeStruct((NC, npad, dh), f32),
        mesh=mesh,
        scratch_types=common_scratch + [
            pltpu.VMEM_SHARED((npad, dh), f32),
            pltpu.VMEM_SHARED((npad, dh), f32),
        ] + sem_scratch,
        compiler_params=sc_params,
    )

    bl0r = bl0.reshape(1, dh)
    bl1r = bl1.reshape(1, dh)
    bl2r = bl2.reshape(1, dh)

    z0, r0 = proj(x, Wl0.T, Wr0.T, bl0r)
    part0, cntp = agg_cnt(z0, ei, ones, zeros, zerosc)
    z1, r1 = mid(part0, cntp, r0, Wl1.T, Wr1.T, bl1r)
    part1 = agg(z1, ei, zeros)
    z2, r2 = mid(part1, cntp, r1, Wl2.T, Wr2.T, bl2r)
    part2 = agg(z2, ei, zeros)
    return fin(part2, cntp, r2)


# speedup vs baseline: 2.0947x; 1.0104x over previous
import functools

import jax
import jax.numpy as jnp
from jax import lax
from jax.experimental import pallas as pl
from jax.experimental.pallas import tpu as pltpu, tpu_sc as plsc

NC, NS = 2, 16
NW = NC * NS
CH = 128
CW = 16


def _tc_proj(x_ref, wlT_ref, wrT_ref, bl_ref, z_ref, r_ref):
    xv = x_ref[...]
    z_ref[...] = jnp.dot(xv, wlT_ref[...], preferred_element_type=jnp.float32)
    r_ref[...] = (jnp.dot(xv, wrT_ref[...], preferred_element_type=jnp.float32)
                  + bl_ref[...])


def _tc_mid(part_ref, cntp_ref, r_ref, wlT_ref, wrT_ref, bl_ref, z_ref, rn_ref):
    cnt = cntp_ref[0, :, 0:1] + cntp_ref[1, :, 0:1]
    inv = 1.0 / jnp.maximum(cnt, 1.0)
    agg = part_ref[0] + part_ref[1]
    h = jnp.maximum(agg * inv + r_ref[...], 0.0)
    z_ref[...] = jnp.dot(h, wlT_ref[...], preferred_element_type=jnp.float32)
    rn_ref[...] = (jnp.dot(h, wrT_ref[...], preferred_element_type=jnp.float32)
                   + bl_ref[...])


def _tc_fin(part_ref, cntp_ref, r_ref, o_ref):
    cnt = cntp_ref[0, :, 0:1] + cntp_ref[1, :, 0:1]
    inv = 1.0 / jnp.maximum(cnt, 1.0)
    agg = part_ref[0] + part_ref[1]
    o_ref[...] = agg * inv + r_ref[...]


NBUF = 8
LOOK = 4


def _sc_agg(n, npad, k, dh, with_cnt, *refs):
    if with_cnt:
        (z_hbm, ei_hbm, ones_hbm, zeros_hbm, zerosc_hbm,
         part_hbm, cntp_hbm,
         src_v, dst_v, rows_v, ones_v, acc_sh, z_sh, cnt_sh, *sems) = refs
    else:
        (z_hbm, ei_hbm, zeros_hbm,
         part_hbm,
         src_v, dst_v, rows_v, acc_sh, z_sh, *sems) = refs
    gsems = sems[:NBUF]
    ssems = sems[NBUF:2 * NBUF]
    csem = sems[2 * NBUF]
    c = lax.axis_index("c")
    s = lax.axis_index("s")
    blk = c * NS + s
    zrows = npad // NS
    zr = s * zrows
    ct = ei_hbm.shape[1]
    kbase = ct // NW
    extras = ct - NW * kbase
    pltpu.sync_copy(zeros_hbm.at[pl.ds(zr, zrows)], acc_sh.at[pl.ds(zr, zrows)])
    pltpu.sync_copy(z_hbm.at[pl.ds(zr, zrows)], z_sh.at[pl.ds(zr, zrows)])
    if with_cnt:
        pltpu.sync_copy(zerosc_hbm.at[pl.ds(zr, zrows)],
                        cnt_sh.at[pl.ds(zr, zrows)])
        pltpu.sync_copy(ones_hbm, ones_v)
    pltpu.sync_copy(ei_hbm.at[0, pl.ds(blk * kbase, kbase)],
                    src_v.at[pl.ds(0, kbase)])
    pltpu.sync_copy(ei_hbm.at[1, pl.ds(blk * kbase, kbase)],
                    dst_v.at[pl.ds(0, kbase)])
    zero16 = jnp.zeros((16,), jnp.int32)
    pad16 = jnp.full((16,), n, jnp.int32)
    for row in range(kbase, k):
        for i in range(CH // 16):
            src_v[row, pl.ds(i * 16, 16)] = zero16
            dst_v[row, pl.ds(i * 16, 16)] = pad16

    @pl.when(blk < extras)
    def _():
        pltpu.sync_copy(ei_hbm.at[0, pl.ds(NW * kbase + blk, 1)],
                        src_v.at[pl.ds(kbase, 1)])
        pltpu.sync_copy(ei_hbm.at[1, pl.ds(NW * kbase + blk, 1)],
                        dst_v.at[pl.ds(kbase, 1)])

    plsc.subcore_barrier()

    for b in range(LOOK):
        pltpu.async_copy(z_sh.at[src_v.at[b]], rows_v.at[b], gsems[b])

    def body(jo, carry):
        for b8 in range(NBUF):
            j = jo * NBUF + b8
            b = b8
            bn = (b8 + LOOK) % NBUF
            pltpu.make_async_copy(z_sh.at[src_v.at[j]], rows_v.at[b],
                                  gsems[b]).wait()
            pltpu.async_copy(rows_v.at[b], acc_sh.at[dst_v.at[j]],
                             ssems[b], add=True)
            if with_cnt:
                pltpu.async_copy(ones_v, cnt_sh.at[dst_v.at[j]],
                                 csem, add=True)
            jn = j + LOOK

            @pl.when(jnp.logical_and(jn < k, j >= LOOK))
            def _():
                pltpu.make_async_copy(rows_v.at[bn], acc_sh.at[dst_v.at[j]],
                                      ssems[bn]).wait()

            @pl.when(jn < k)
            def _():
                pltpu.async_copy(z_sh.at[src_v.at[jn]], rows_v.at[bn],
                                 gsems[bn])
        return carry

    lax.fori_loop(0, k // NBUF, body, 0)
    for b in range(NBUF):
        pltpu.make_async_copy(rows_v.at[b], acc_sh.at[dst_v.at[0]],
                              ssems[b]).wait()
    if with_cnt:
        def cdrain(j, carry):
            pltpu.make_async_copy(ones_v, cnt_sh.at[dst_v.at[0]],
                                  csem).wait()
            return carry
        lax.fori_loop(0, k, cdrain, 0)
    plsc.subcore_barrier()
    pltpu.sync_copy(acc_sh.at[pl.ds(zr, zrows)],
                    part_hbm.at[c, pl.ds(zr, zrows)])
    if with_cnt:
        pltpu.sync_copy(cnt_sh.at[pl.ds(zr, zrows)],
                        cntp_hbm.at[c, pl.ds(zr, zrows)])


def kernel(x, edge_index, Wl0, bl0, Wr0, Wl1, bl1, Wr1, Wl2, bl2, Wr2):
    n, d_in = x.shape
    dh = Wl0.shape[0]
    e = edge_index.shape[1]
    ct = e // CH
    k = ct // NW + (1 if ct % NW else 0)
    k = -(-k // NBUF) * NBUF
    npad = NS * (-(-(n + 1) // (NS * 8)) * 8)

    ei = edge_index.reshape(2, ct, CH)
    ones = jnp.ones((CH, CW), jnp.float32)
    zeros = jnp.zeros((npad, dh), jnp.float32)
    zerosc = jnp.zeros((npad, CW), jnp.float32)
    f32 = jnp.float32

    rb = 2000
    gr = n // rb
    rowspec = lambda w: pl.BlockSpec((rb, w), lambda i: (i, 0))
    pairspec = lambda w: pl.BlockSpec((NC, rb, w), lambda i: (0, i, 0))
    fullspec = lambda a, b: pl.BlockSpec((a, b), lambda i: (0, 0))
    proj = pl.pallas_call(
        _tc_proj,
        grid=(gr,),
        in_specs=[rowspec(d_in), fullspec(d_in, dh), fullspec(d_in, dh),
                  fullspec(1, dh)],
        out_specs=(rowspec(dh), rowspec(dh)),
        out_shape=(jax.ShapeDtypeStruct((npad, dh), f32),
                   jax.ShapeDtypeStruct((n, dh), f32)),
    )
    mid = pl.pallas_call(
        _tc_mid,
        grid=(gr,),
        in_specs=[pairspec(dh), pairspec(CW), rowspec(dh), fullspec(dh, dh),
                  fullspec(dh, dh), fullspec(1, dh)],
        out_specs=(rowspec(dh), rowspec(dh)),
        out_shape=(jax.ShapeDtypeStruct((npad, dh), f32),
                   jax.ShapeDtypeStruct((n, dh), f32)),
    )
    fin = pl.pallas_call(
        _tc_fin,
        grid=(gr,),
        in_specs=[pairspec(dh), pairspec(CW), rowspec(dh)],
        out_specs=rowspec(dh),
        out_shape=jax.ShapeDtypeStruct((n, dh), f32),
    )

    mesh = plsc.VectorSubcoreMesh(core_axis_name="c", subcore_axis_name="s",
                                  num_cores=NC, num_subcores=NS)
    sc_params = pltpu.CompilerParams(use_tc_tiling_on_sc=False)
    common_scratch = [
        pltpu.VMEM((k, CH), jnp.int32),
        pltpu.VMEM((k, CH), jnp.int32),
        pltpu.VMEM((NBUF, CH, dh), f32),
    ]
    sem_scratch = [pltpu.SemaphoreType.DMA] * (2 * NBUF + 1)
    agg_cnt = pl.kernel(
        functools.partial(_sc_agg, n, npad, k, dh, True),
        out_type=(jax.ShapeDtypeStruct((NC, npad, dh), f32),
                  jax.ShapeDtypeStruct((NC, npad, CW), f32)),
        mesh=mesh,
        scratch_types=common_scratch + [
            pltpu.VMEM((CH, CW), f32),
            pltpu.VMEM_SHARED((npad, dh), f32),
            pltpu.VMEM_SHARED((npad, dh), f32),
            pltpu.VMEM_SHARED((npad, CW), f32),
        ] + sem_scratch,
        compiler_params=sc_params,
    )
    agg = pl.kernel(
        functools.partial(_sc_agg, n, npad, k, dh, False),
        out_type=jax.ShapeDtypeStruct((NC, npad, dh), f32),
        mesh=mesh,
        scratch_types=common_scratch + [
            pltpu.VMEM_SHARED((npad, dh), f32),
            pltpu.VMEM_SHARED((npad, dh), f32),
        ] + sem_scratch,
        compiler_params=sc_params,
    )

    bl0r = bl0.reshape(1, dh)
    bl1r = bl1.reshape(1, dh)
    bl2r = bl2.reshape(1, dh)

    z0, r0 = proj(x, Wl0.T, Wr0.T, bl0r)
    part0, cntp = agg_cnt(z0, ei, ones, zeros, zerosc)
    z1, r1 = mid(part0, cntp, r0, Wl1.T, Wr1.T, bl1r)
    part1 = agg(z1, ei, zeros)
    z2, r2 = mid(part1, cntp, r1, Wl2.T, Wr2.T, bl2r)
    part2 = agg(z2, ei, zeros)
    return fin(part2, cntp, r2)


# speedup vs baseline: 2.6733x; 1.2762x over previous
import functools

import jax
import jax.numpy as jnp
from jax import lax
from jax.experimental import pallas as pl
from jax.experimental.pallas import tpu as pltpu, tpu_sc as plsc

NC, NS = 2, 16
NW = NC * NS
CH = 128
CW = 32


def _tc_proj(x_ref, wl4_ref, wr4_ref, bl4_ref, z_ref, r_ref):
    n, d_in = x_ref.shape
    xv = jnp.reshape(x_ref[...], (n // 4, 4 * d_in))
    zv = jnp.dot(xv, wl4_ref[...], preferred_element_type=jnp.float32)
    z_ref[0:zv.shape[0], :] = zv
    r_ref[...] = (jnp.dot(xv, wr4_ref[...], preferred_element_type=jnp.float32)
                  + bl4_ref[...])


def _tc_mid(part_ref, cntp_ref, r_ref, wl4_ref, wr4_ref, bl4_ref, z_ref, rn_ref):
    cnt = cntp_ref[0] + cntp_ref[1]
    inv = 1.0 / jnp.maximum(cnt, 1.0)
    agg = part_ref[0] + part_ref[1]
    nfn = r_ref.shape[0]
    h = jnp.maximum(agg[0:nfn] * inv[0:nfn] + r_ref[...], 0.0)
    z_ref[0:nfn, :] = jnp.dot(h, wl4_ref[...], preferred_element_type=jnp.float32)
    rn_ref[...] = (jnp.dot(h, wr4_ref[...], preferred_element_type=jnp.float32)
                   + bl4_ref[...])


def _tc_fin(part_ref, cntp_ref, r_ref, o_ref):
    nfn = r_ref.shape[0]
    cnt = cntp_ref[0] + cntp_ref[1]
    inv = 1.0 / jnp.maximum(cnt, 1.0)
    agg = part_ref[0] + part_ref[1]
    o_ref[...] = agg[0:nfn] * inv[0:nfn] + r_ref[...]


NBUF = 8
LOOK = 4


def _sc_agg(n, npad, k, dh, with_cnt, *refs):
    if with_cnt:
        (z_hbm, ei_hbm, ones_hbm, zeros_hbm, zerosc_hbm,
         part_hbm, cntp_hbm,
         src_v, dst_v, rows_v, ones_v, acc_sh, z_sh, cnt_sh, *sems) = refs
    else:
        (z_hbm, ei_hbm, zeros_hbm,
         part_hbm,
         src_v, dst_v, rows_v, acc_sh, z_sh, *sems) = refs
    gsems = sems[:NBUF]
    ssems = sems[NBUF:2 * NBUF]
    csem = sems[2 * NBUF]
    c = lax.axis_index("c")
    s = lax.axis_index("s")
    blk = c * NS + s
    zrows = npad // NS
    zr = s * zrows
    ct = ei_hbm.shape[1]
    kbase = ct // NW
    extras = ct - NW * kbase
    pltpu.sync_copy(zeros_hbm.at[pl.ds(zr, zrows)], acc_sh.at[pl.ds(zr, zrows)])
    pltpu.sync_copy(z_hbm.at[pl.ds(zr, zrows)], z_sh.at[pl.ds(zr, zrows)])
    if with_cnt:
        pltpu.sync_copy(zerosc_hbm.at[pl.ds(zr, zrows)],
                        cnt_sh.at[pl.ds(zr, zrows)])
        pltpu.sync_copy(ones_hbm, ones_v)
    pltpu.sync_copy(ei_hbm.at[0, pl.ds(blk * kbase, kbase)],
                    src_v.at[pl.ds(0, kbase)])
    pltpu.sync_copy(ei_hbm.at[1, pl.ds(blk * kbase, kbase)],
                    dst_v.at[pl.ds(0, kbase)])
    zero16 = jnp.zeros((16,), jnp.int32)
    pad16 = jnp.full((16,), n, jnp.int32)
    for row in range(kbase, k):
        for i in range(CH // 16):
            src_v[row, pl.ds(i * 16, 16)] = zero16
            dst_v[row, pl.ds(i * 16, 16)] = pad16

    @pl.when(blk < extras)
    def _():
        pltpu.sync_copy(ei_hbm.at[0, pl.ds(NW * kbase + blk, 1)],
                        src_v.at[pl.ds(kbase, 1)])
        pltpu.sync_copy(ei_hbm.at[1, pl.ds(NW * kbase + blk, 1)],
                        dst_v.at[pl.ds(kbase, 1)])

    plsc.subcore_barrier()

    for b in range(LOOK):
        pltpu.async_copy(z_sh.at[src_v.at[b]], rows_v.at[b], gsems[b])

    def body(jo, carry):
        for b8 in range(NBUF):
            j = jo * NBUF + b8
            b = b8
            bn = (b8 + LOOK) % NBUF
            pltpu.make_async_copy(z_sh.at[src_v.at[j]], rows_v.at[b],
                                  gsems[b]).wait()
            pltpu.async_copy(rows_v.at[b], acc_sh.at[dst_v.at[j]],
                             ssems[b], add=True)
            if with_cnt:
                pltpu.async_copy(ones_v, cnt_sh.at[dst_v.at[j]],
                                 csem, add=True)
            jn = j + LOOK

            @pl.when(jnp.logical_and(jn < k, j >= LOOK))
            def _():
                pltpu.make_async_copy(rows_v.at[bn], acc_sh.at[dst_v.at[j]],
                                      ssems[bn]).wait()

            @pl.when(jn < k)
            def _():
                pltpu.async_copy(z_sh.at[src_v.at[jn]], rows_v.at[bn],
                                 gsems[bn])
        return carry

    lax.fori_loop(0, k // NBUF, body, 0)
    for b in range(NBUF):
        pltpu.make_async_copy(rows_v.at[b], acc_sh.at[dst_v.at[0]],
                              ssems[b]).wait()
    if with_cnt:
        def cdrain(j, carry):
            pltpu.make_async_copy(ones_v, cnt_sh.at[dst_v.at[0]],
                                  csem).wait()
            return carry
        lax.fori_loop(0, k, cdrain, 0)
    plsc.subcore_barrier()
    pltpu.sync_copy(acc_sh.at[pl.ds(zr, zrows)],
                    part_hbm.at[c, pl.ds(zr, zrows)])
    if with_cnt:
        pltpu.sync_copy(cnt_sh.at[pl.ds(zr, zrows)],
                        cntp_hbm.at[c, pl.ds(zr, zrows)])


def kernel(x, edge_index, Wl0, bl0, Wr0, Wl1, bl1, Wr1, Wl2, bl2, Wr2):
    n, d_in = x.shape
    dh = Wl0.shape[0]
    e = edge_index.shape[1]
    ct = e // CH
    k = ct // NW + (1 if ct % NW else 0)
    k = -(-k // NBUF) * NBUF
    npad = NS * (-(-(n + 1) // (NS * 8)) * 8)

    ei = edge_index.reshape(2, ct, CH)
    ones = jnp.ones((CH, CW), jnp.float32)
    zeros = jnp.zeros((npad, dh), jnp.float32)
    zerosc = jnp.zeros((npad, CW), jnp.float32)
    f32 = jnp.float32

    nf = npad * dh // 128
    nfn = n * dh // 128
    proj = pl.pallas_call(
        _tc_proj,
        out_shape=(jax.ShapeDtypeStruct((nf, 128), f32),
                   jax.ShapeDtypeStruct((nfn, 128), f32)),
    )
    mid = pl.pallas_call(
        _tc_mid,
        out_shape=(jax.ShapeDtypeStruct((nf, 128), f32),
                   jax.ShapeDtypeStruct((nfn, 128), f32)),
    )
    fin = pl.pallas_call(
        _tc_fin,
        out_shape=jax.ShapeDtypeStruct((nfn, 128), f32),
    )

    mesh = plsc.VectorSubcoreMesh(core_axis_name="c", subcore_axis_name="s",
                                  num_cores=NC, num_subcores=NS)
    sc_params = pltpu.CompilerParams(use_tc_tiling_on_sc=False)
    common_scratch = [
        pltpu.VMEM((k, CH), jnp.int32),
        pltpu.VMEM((k, CH), jnp.int32),
        pltpu.VMEM((NBUF, CH, dh), f32),
    ]
    sem_scratch = [pltpu.SemaphoreType.DMA] * (2 * NBUF + 1)
    agg_cnt = pl.kernel(
        functools.partial(_sc_agg, n, npad, k, dh, True),
        out_type=(jax.ShapeDtypeStruct((NC, npad, dh), f32),
                  jax.ShapeDtypeStruct((NC, npad, CW), f32)),
        mesh=mesh,
        scratch_types=common_scratch + [
            pltpu.VMEM((CH, CW), f32),
            pltpu.VMEM_SHARED((npad, dh), f32),
            pltpu.VMEM_SHARED((npad, dh), f32),
            pltpu.VMEM_SHARED((npad, CW), f32),
        ] + sem_scratch,
        compiler_params=sc_params,
    )
    agg = pl.kernel(
        functools.partial(_sc_agg, n, npad, k, dh, False),
        out_type=jax.ShapeDtypeStruct((NC, npad, dh), f32),
        mesh=mesh,
        scratch_types=common_scratch + [
            pltpu.VMEM_SHARED((npad, dh), f32),
            pltpu.VMEM_SHARED((npad, dh), f32),
        ] + sem_scratch,
        compiler_params=sc_params,
    )

    eye4 = jnp.eye(4, dtype=jnp.float32)
    wl4_1 = jnp.kron(eye4, Wl1.T)
    wr4_1 = jnp.kron(eye4, Wr1.T)
    wl4_2 = jnp.kron(eye4, Wl2.T)
    wr4_2 = jnp.kron(eye4, Wr2.T)
    bl0r = jnp.tile(bl0, 4).reshape(1, 128)
    bl1r = jnp.tile(bl1, 4).reshape(1, 128)
    bl2r = jnp.tile(bl2, 4).reshape(1, 128)

    wl4_0 = jnp.kron(eye4, Wl0.T)
    wr4_0 = jnp.kron(eye4, Wr0.T)
    z0f, r0f = proj(x, wl4_0, wr4_0, bl0r)
    part0, cntp = agg_cnt(z0f.reshape(npad, dh), ei, ones, zeros, zerosc)
    z1f, r1f = mid(part0.reshape(NC, nf, 128), cntp.reshape(NC, nf, 128),
                   r0f, wl4_1, wr4_1, bl1r)
    part1 = agg(z1f.reshape(npad, dh), ei, zeros)
    z2f, r2f = mid(part1.reshape(NC, nf, 128), cntp.reshape(NC, nf, 128),
                   r1f, wl4_2, wr4_2, bl2r)
    part2 = agg(z2f.reshape(npad, dh), ei, zeros)
    of = fin(part2.reshape(NC, nf, 128), cntp.reshape(NC, nf, 128), r2f)
    return of.reshape(n, dh)


# speedup vs baseline: 2.6758x; 1.0009x over previous
import functools

import jax
import jax.numpy as jnp
from jax import lax
from jax.experimental import pallas as pl
from jax.experimental.pallas import tpu as pltpu, tpu_sc as plsc

NC, NS = 2, 16
NW = NC * NS
CH = 128
CW = 32


def _tc_proj(x_ref, wl4_ref, wr4_ref, bl4_ref, z_ref, r_ref):
    n, d_in = x_ref.shape
    xv = jnp.reshape(x_ref[...], (n // 4, 4 * d_in))
    zv = jnp.dot(xv, wl4_ref[...], preferred_element_type=jnp.float32)
    z_ref[0:zv.shape[0], :] = zv
    r_ref[...] = (jnp.dot(xv, wr4_ref[...], preferred_element_type=jnp.float32)
                  + bl4_ref[...])


def _tc_mid(part_ref, cntp_ref, r_ref, wl4_ref, wr4_ref, bl4_ref, z_ref, rn_ref):
    cnt = cntp_ref[0] + cntp_ref[1]
    inv = 1.0 / jnp.maximum(cnt, 1.0)
    agg = part_ref[0] + part_ref[1]
    nfn = r_ref.shape[0]
    h = jnp.maximum(agg[0:nfn] * inv[0:nfn] + r_ref[...], 0.0)
    z_ref[0:nfn, :] = jnp.dot(h, wl4_ref[...], preferred_element_type=jnp.float32)
    rn_ref[...] = (jnp.dot(h, wr4_ref[...], preferred_element_type=jnp.float32)
                   + bl4_ref[...])


def _tc_fin(part_ref, cntp_ref, r_ref, o_ref):
    nfn = r_ref.shape[0]
    cnt = cntp_ref[0] + cntp_ref[1]
    inv = 1.0 / jnp.maximum(cnt, 1.0)
    agg = part_ref[0] + part_ref[1]
    o_ref[...] = agg[0:nfn] * inv[0:nfn] + r_ref[...]


NBUF = 8
LOOK = 6


def _sc_agg(n, npad, k, dh, with_cnt, *refs):
    if with_cnt:
        (z_hbm, ei_hbm, ones_hbm, zeros_hbm, zerosc_hbm,
         part_hbm, cntp_hbm,
         src_v, dst_v, rows_v, ones_v, acc_sh, z_sh, cnt_sh, *sems) = refs
    else:
        (z_hbm, ei_hbm, zeros_hbm,
         part_hbm,
         src_v, dst_v, rows_v, acc_sh, z_sh, *sems) = refs
    gsems = sems[:NBUF]
    ssems = sems[NBUF:2 * NBUF]
    csem = sems[2 * NBUF]
    c = lax.axis_index("c")
    s = lax.axis_index("s")
    blk = c * NS + s
    zrows = npad // NS
    zr = s * zrows
    ct = ei_hbm.shape[1]
    kbase = ct // NW
    extras = ct - NW * kbase
    pltpu.sync_copy(zeros_hbm.at[pl.ds(zr, zrows)], acc_sh.at[pl.ds(zr, zrows)])
    pltpu.sync_copy(z_hbm.at[pl.ds(zr, zrows)], z_sh.at[pl.ds(zr, zrows)])
    if with_cnt:
        pltpu.sync_copy(zerosc_hbm.at[pl.ds(zr, zrows)],
                        cnt_sh.at[pl.ds(zr, zrows)])
        pltpu.sync_copy(ones_hbm, ones_v)
    pltpu.sync_copy(ei_hbm.at[0, pl.ds(blk * kbase, kbase)],
                    src_v.at[pl.ds(0, kbase)])
    pltpu.sync_copy(ei_hbm.at[1, pl.ds(blk * kbase, kbase)],
                    dst_v.at[pl.ds(0, kbase)])
    zero16 = jnp.zeros((16,), jnp.int32)
    pad16 = jnp.full((16,), n, jnp.int32)
    for row in range(kbase, k):
        for i in range(CH // 16):
            src_v[row, pl.ds(i * 16, 16)] = zero16
            dst_v[row, pl.ds(i * 16, 16)] = pad16

    @pl.when(blk < extras)
    def _():
        pltpu.sync_copy(ei_hbm.at[0, pl.ds(NW * kbase + blk, 1)],
                        src_v.at[pl.ds(kbase, 1)])
        pltpu.sync_copy(ei_hbm.at[1, pl.ds(NW * kbase + blk, 1)],
                        dst_v.at[pl.ds(kbase, 1)])

    plsc.subcore_barrier()

    for b in range(LOOK):
        pltpu.async_copy(z_sh.at[src_v.at[b]], rows_v.at[b], gsems[b])

    def body(jo, carry):
        for b8 in range(NBUF):
            j = jo * NBUF + b8
            b = b8
            bn = (b8 + LOOK) % NBUF
            pltpu.make_async_copy(z_sh.at[src_v.at[j]], rows_v.at[b],
                                  gsems[b]).wait()
            pltpu.async_copy(rows_v.at[b], acc_sh.at[dst_v.at[j]],
                             ssems[b], add=True)
            if with_cnt:
                pltpu.async_copy(ones_v, cnt_sh.at[dst_v.at[j]],
                                 csem, add=True)
            jn = j + LOOK

            @pl.when(jnp.logical_and(jn < k, j >= NBUF - LOOK))
            def _():
                pltpu.make_async_copy(rows_v.at[bn], acc_sh.at[dst_v.at[j]],
                                      ssems[bn]).wait()

            @pl.when(jn < k)
            def _():
                pltpu.async_copy(z_sh.at[src_v.at[jn]], rows_v.at[bn],
                                 gsems[bn])
        return carry

    lax.fori_loop(0, k // NBUF, body, 0)
    for b in range(NBUF):
        pltpu.make_async_copy(rows_v.at[b], acc_sh.at[dst_v.at[0]],
                              ssems[b]).wait()
    if with_cnt:
        def cdrain(j, carry):
            pltpu.make_async_copy(ones_v, cnt_sh.at[dst_v.at[0]],
                                  csem).wait()
            return carry
        lax.fori_loop(0, k, cdrain, 0)
    plsc.subcore_barrier()
    pltpu.sync_copy(acc_sh.at[pl.ds(zr, zrows)],
                    part_hbm.at[c, pl.ds(zr, zrows)])
    if with_cnt:
        pltpu.sync_copy(cnt_sh.at[pl.ds(zr, zrows)],
                        cntp_hbm.at[c, pl.ds(zr, zrows)])


def kernel(x, edge_index, Wl0, bl0, Wr0, Wl1, bl1, Wr1, Wl2, bl2, Wr2):
    n, d_in = x.shape
    dh = Wl0.shape[0]
    e = edge_index.shape[1]
    ct = e // CH
    k = ct // NW + (1 if ct % NW else 0)
    k = -(-k // NBUF) * NBUF
    npad = NS * (-(-(n + 1) // (NS * 8)) * 8)

    ei = edge_index.reshape(2, ct, CH)
    ones = jnp.ones((CH, CW), jnp.float32)
    zeros = jnp.zeros((npad, dh), jnp.float32)
    zerosc = jnp.zeros((npad, CW), jnp.float32)
    f32 = jnp.float32

    nf = npad * dh // 128
    nfn = n * dh // 128
    proj = pl.pallas_call(
        _tc_proj,
        out_shape=(jax.ShapeDtypeStruct((nf, 128), f32),
                   jax.ShapeDtypeStruct((nfn, 128), f32)),
    )
    mid = pl.pallas_call(
        _tc_mid,
        out_shape=(jax.ShapeDtypeStruct((nf, 128), f32),
                   jax.ShapeDtypeStruct((nfn, 128), f32)),
    )
    fin = pl.pallas_call(
        _tc_fin,
        out_shape=jax.ShapeDtypeStruct((nfn, 128), f32),
    )

    mesh = plsc.VectorSubcoreMesh(core_axis_name="c", subcore_axis_name="s",
                                  num_cores=NC, num_subcores=NS)
    sc_params = pltpu.CompilerParams(use_tc_tiling_on_sc=False)
    common_scratch = [
        pltpu.VMEM((k, CH), jnp.int32),
        pltpu.VMEM((k, CH), jnp.int32),
        pltpu.VMEM((NBUF, CH, dh), f32),
    ]
    sem_scratch = [pltpu.SemaphoreType.DMA] * (2 * NBUF + 1)
    agg_cnt = pl.kernel(
        functools.partial(_sc_agg, n, npad, k, dh, True),
        out_type=(jax.ShapeDtypeStruct((NC, npad, dh), f32),
                  jax.ShapeDtypeStruct((NC, npad, CW), f32)),
        mesh=mesh,
        scratch_types=common_scratch + [
            pltpu.VMEM((CH, CW), f32),
            pltpu.VMEM_SHARED((npad, dh), f32),
            pltpu.VMEM_SHARED((npad, dh), f32),
            pltpu.VMEM_SHARED((npad, CW), f32),
        ] + sem_scratch,
        compiler_params=sc_params,
    )
    agg = pl.kernel(
        functools.partial(_sc_agg, n, npad, k, dh, False),
        out_type=jax.ShapeDtypeStruct((NC, npad, dh), f32),
        mesh=mesh,
        scratch_types=common_scratch + [
            pltpu.VMEM_SHARED((npad, dh), f32),
            pltpu.VMEM_SHARED((npad, dh), f32),
        ] + sem_scratch,
        compiler_params=sc_params,
    )

    eye4 = jnp.eye(4, dtype=jnp.float32)
    wl4_1 = jnp.kron(eye4, Wl1.T)
    wr4_1 = jnp.kron(eye4, Wr1.T)
    wl4_2 = jnp.kron(eye4, Wl2.T)
    wr4_2 = jnp.kron(eye4, Wr2.T)
    bl0r = jnp.tile(bl0, 4).reshape(1, 128)
    bl1r = jnp.tile(bl1, 4).reshape(1, 128)
    bl2r = jnp.tile(bl2, 4).reshape(1, 128)

    wl4_0 = jnp.kron(eye4, Wl0.T)
    wr4_0 = jnp.kron(eye4, Wr0.T)
    z0f, r0f = proj(x, wl4_0, wr4_0, bl0r)
    part0, cntp = agg_cnt(z0f.reshape(npad, dh), ei, ones, zeros, zerosc)
    z1f, r1f = mid(part0.reshape(NC, nf, 128), cntp.reshape(NC, nf, 128),
                   r0f, wl4_1, wr4_1, bl1r)
    part1 = agg(z1f.reshape(npad, dh), ei, zeros)
    z2f, r2f = mid(part1.reshape(NC, nf, 128), cntp.reshape(NC, nf, 128),
                   r1f, wl4_2, wr4_2, bl2r)
    part2 = agg(z2f.reshape(npad, dh), ei, zeros)
    of = fin(part2.reshape(NC, nf, 128), cntp.reshape(NC, nf, 128), r2f)
    return of.reshape(n, dh)
